# Initial kernel scaffold; baseline (speedup 1.0000x reference)
import jax
import jax.numpy as jnp
from jax.experimental import pallas as pl


def kernel(x, guide, cv1_w, cv1_bn_g, cv1_bn_b, cv1_bn_m, cv1_bn_v, cv2_w, cv2_bn_g, cv2_bn_b, cv2_bn_m, cv2_bn_v, m0_w1, m0_bn1_g, m0_bn1_b, m0_bn1_m, m0_bn1_v, m0_w2, m0_bn2_g, m0_bn2_b, m0_bn2_m, m0_bn2_v, m1_w1, m1_bn1_g, m1_bn1_b, m1_bn1_m, m1_bn1_v, m1_w2, m1_bn2_g, m1_bn2_b, m1_bn2_m, m1_bn2_v, m2_w1, m2_bn1_g, m2_bn1_b, m2_bn1_m, m2_bn1_v, m2_w2, m2_bn2_g, m2_bn2_b, m2_bn2_m, m2_bn2_v, gl_w, gl_b, attn_bias, proj_w, proj_bn_g, proj_bn_b, proj_bn_m, proj_bn_v):
    raise NotImplementedError("write your pallas kernel here")


# speedup vs baseline: 1.9596x; 1.9596x over previous
import jax
import jax.numpy as jnp
import numpy as np
from jax.experimental import pallas as pl
from jax.experimental.pallas import tpu as pltpu

_BN_EPS = 1e-5


def _fold_bn(g, b, m, v, eps=_BN_EPS):
    s = g / jnp.sqrt(v + eps)
    return s, b - m * s


def _conv_taps3(w):
    cout, cin = w.shape[0], w.shape[1]
    return jnp.transpose(w, (2, 0, 3, 1)).reshape(3, cout, 3 * cin)


def _edge_masks(H, W):
    HW = H * W
    p = np.arange(HW)
    rows = np.zeros((8, HW), np.float32)
    rows[0] = p % W != 0
    rows[1] = p % W != W - 1
    rows[2] = p >= W
    rows[3] = p < HW - W
    return rows


def _rotl(a, k, n):
    k %= n
    if k == 0:
        return a
    return jnp.concatenate([a[:, k:], a[:, :k]], axis=1)


def kernel(x, guide, cv1_w, cv1_bn_g, cv1_bn_b, cv1_bn_m, cv1_bn_v,
           cv2_w, cv2_bn_g, cv2_bn_b, cv2_bn_m, cv2_bn_v,
           m0_w1, m0_bn1_g, m0_bn1_b, m0_bn1_m, m0_bn1_v,
           m0_w2, m0_bn2_g, m0_bn2_b, m0_bn2_m, m0_bn2_v,
           m1_w1, m1_bn1_g, m1_bn1_b, m1_bn1_m, m1_bn1_v,
           m1_w2, m1_bn2_g, m1_bn2_b, m1_bn2_m, m1_bn2_v,
           m2_w1, m2_bn1_g, m2_bn1_b, m2_bn1_m, m2_bn1_v,
           m2_w2, m2_bn2_g, m2_bn2_b, m2_bn2_m, m2_bn2_v,
           gl_w, gl_b, attn_bias,
           proj_w, proj_bn_g, proj_bn_b, proj_bn_m, proj_bn_v):
    N, c1, H, W = x.shape
    T, gc = guide.shape[1], guide.shape[2]
    HW = H * W
    c = cv1_w.shape[0] // 2
    c2 = cv2_w.shape[0]
    nh = attn_bias.shape[0]
    hc = c // nh
    bf16 = jnp.bfloat16

    def col(v):
        return v.reshape(-1, 1)

    c1s, c1b = _fold_bn(cv1_bn_g, cv1_bn_b, cv1_bn_m, cv1_bn_v)
    c2s, c2b = _fold_bn(cv2_bn_g, cv2_bn_b, cv2_bn_m, cv2_bn_v)
    convs = []
    for w1, g1, b1, mm1, v1, w2, g2, b2, mm2, v2 in (
            (m0_w1, m0_bn1_g, m0_bn1_b, m0_bn1_m, m0_bn1_v,
             m0_w2, m0_bn2_g, m0_bn2_b, m0_bn2_m, m0_bn2_v),
            (m1_w1, m1_bn1_g, m1_bn1_b, m1_bn1_m, m1_bn1_v,
             m1_w2, m1_bn2_g, m1_bn2_b, m1_bn2_m, m1_bn2_v),
            (m2_w1, m2_bn1_g, m2_bn1_b, m2_bn1_m, m2_bn1_v,
             m2_w2, m2_bn2_g, m2_bn2_b, m2_bn2_m, m2_bn2_v)):
        s1, bb1 = _fold_bn(g1, b1, mm1, v1)
        s2, bb2 = _fold_bn(g2, b2, mm2, v2)
        convs += [_conv_taps3(w1).astype(bf16), col(s1), col(bb1),
                  _conv_taps3(w2).astype(bf16), col(s2), col(bb2)]
    ps_, pb_ = _fold_bn(proj_bn_g, proj_bn_b, proj_bn_m, proj_bn_v)
    ab_pad = jnp.zeros((8, 128), jnp.float32).at[:nh, :].set(
        attn_bias.reshape(nh, 1))
    masks = jnp.asarray(_edge_masks(H, W), dtype=bf16)

    operands = ([x.reshape(N, c1, HW), guide, masks,
                 cv1_w[:, :, 0, 0].astype(bf16), col(c1s), col(c1b)]
                + convs
                + [_conv_taps3(proj_w).astype(bf16), col(ps_), col(pb_),
                   gl_w.T.astype(bf16), gl_b.reshape(1, -1), ab_pad,
                   cv2_w[:, :, 0, 0].astype(bf16), col(c2s), col(c2b)])

    def body(*refs):
        (x_ref, g_ref, m_ref, c1w, c1sr, c1br) = refs[0:6]
        conv_refs = refs[6:24]
        (pw, psr, pbr, glw, glb, ab, c2w, c2sr, c2br) = refs[24:33]
        o_ref = refs[33]

        msk = m_ref[...]
        mxm, mxp, myt, myb = (msk[0:1], msk[1:2], msk[2:3], msk[3:4])

        def conv3(xb, w_ref):
            x3 = jnp.concatenate(
                [_rotl(xb, -1, HW) * mxm, xb, _rotl(xb, 1, HW) * mxp], axis=0)
            acc = jnp.dot(w_ref[1], x3, preferred_element_type=jnp.float32)
            acc = acc + jnp.dot(w_ref[0], _rotl(x3, -W, HW) * myt,
                                preferred_element_type=jnp.float32)
            acc = acc + jnp.dot(w_ref[2], _rotl(x3, W, HW) * myb,
                                preferred_element_type=jnp.float32)
            return acc

        def affine(acc, s_ref, b_ref, act):
            y = acc * s_ref[...] + b_ref[...]
            if act:
                y = y * jax.nn.sigmoid(y)
            return y

        xb = x_ref[0].astype(bf16)
        y = affine(jnp.dot(c1w[...], xb, preferred_element_type=jnp.float32),
                   c1sr, c1br, True)
        ys = [y[:c].astype(bf16)]
        cur = y[c:].astype(bf16)
        ys.append(cur)
        for i in range(3):
            w1, s1, b1, w2, s2, b2 = conv_refs[6 * i:6 * i + 6]
            t = affine(conv3(cur, w1), s1, b1, True).astype(bf16)
            cur = affine(conv3(t, w2), s2, b2, True).astype(bf16)
            ys.append(cur)

        ge = jnp.dot(g_ref[0].astype(bf16), glw[...],
                     preferred_element_type=jnp.float32) + glb[...]
        geb = ge.astype(bf16)

        proj = affine(conv3(cur, pw), psr, pbr, False)
        rows = []
        for m in range(nh):
            s = jnp.dot(geb[:, m * hc:(m + 1) * hc],
                        cur[m * hc:(m + 1) * hc],
                        preferred_element_type=jnp.float32)
            rows.append(jnp.max(s, axis=0, keepdims=True))
        aw = jnp.concatenate(rows, axis=0) if nh > 1 else rows[0]
        gate = jax.nn.sigmoid(aw * (1.0 / float(hc) ** 0.5) + ab[0:nh, 0:1])
        att = [(proj[m * hc:(m + 1) * hc] * gate[m:m + 1]).astype(bf16)
               for m in range(nh)]
        ys.append(jnp.concatenate(att, axis=0) if nh > 1 else att[0])

        cat = jnp.concatenate(ys, axis=0)
        o_ref[0] = affine(
            jnp.dot(c2w[...], cat, preferred_element_type=jnp.float32),
            c2sr, c2br, True)

    fixed = lambda *shape: pl.BlockSpec(shape, lambda n: (0,) * len(shape))
    conv_specs = []
    for _ in range(3):
        conv_specs += [fixed(3, c, 3 * c), fixed(c, 1), fixed(c, 1),
                       fixed(3, c, 3 * c), fixed(c, 1), fixed(c, 1)]
    in_specs = ([pl.BlockSpec((1, c1, HW), lambda n: (n, 0, 0)),
                 pl.BlockSpec((1, T, gc), lambda n: (n, 0, 0)),
                 fixed(8, HW),
                 fixed(2 * c, c1), fixed(2 * c, 1), fixed(2 * c, 1)]
                + conv_specs
                + [fixed(3, c, 3 * c), fixed(c, 1), fixed(c, 1),
                   fixed(gc, nh * hc), fixed(1, nh * hc), fixed(8, 128),
                   fixed(c2, (3 + 3) * c), fixed(c2, 1), fixed(c2, 1)])

    out = pl.pallas_call(
        body,
        out_shape=jax.ShapeDtypeStruct((N, c2, HW), jnp.float32),
        grid=(N,),
        in_specs=in_specs,
        out_specs=pl.BlockSpec((1, c2, HW), lambda n: (n, 0, 0)),
        compiler_params=pltpu.CompilerParams(
            dimension_semantics=("parallel",)),
    )(*operands)
    return out.reshape(N, c2, H, W)


# speedup vs baseline: 2.1003x; 1.0718x over previous
import jax
import jax.numpy as jnp
import numpy as np
from jax import lax
from jax.experimental import pallas as pl
from jax.experimental.pallas import tpu as pltpu

_BN_EPS = 1e-5


def _fold_bn(g, b, m, v, eps=_BN_EPS):
    s = g / jnp.sqrt(v + eps)
    return s, b - m * s


def _edge_masks(H, W):
    HW = H * W
    p = np.arange(HW)
    rows = np.zeros((8, HW), np.float32)
    rows[0] = p % W != 0
    rows[1] = p % W != W - 1
    rows[2] = p >= W
    rows[3] = p < HW - W
    return rows


def _rotl(a, k, n):
    k %= n
    if k == 0:
        return a
    return jnp.concatenate([a[:, k:], a[:, :k]], axis=1)


def kernel(x, guide, cv1_w, cv1_bn_g, cv1_bn_b, cv1_bn_m, cv1_bn_v,
           cv2_w, cv2_bn_g, cv2_bn_b, cv2_bn_m, cv2_bn_v,
           m0_w1, m0_bn1_g, m0_bn1_b, m0_bn1_m, m0_bn1_v,
           m0_w2, m0_bn2_g, m0_bn2_b, m0_bn2_m, m0_bn2_v,
           m1_w1, m1_bn1_g, m1_bn1_b, m1_bn1_m, m1_bn1_v,
           m1_w2, m1_bn2_g, m1_bn2_b, m1_bn2_m, m1_bn2_v,
           m2_w1, m2_bn1_g, m2_bn1_b, m2_bn1_m, m2_bn1_v,
           m2_w2, m2_bn2_g, m2_bn2_b, m2_bn2_m, m2_bn2_v,
           gl_w, gl_b, attn_bias,
           proj_w, proj_bn_g, proj_bn_b, proj_bn_m, proj_bn_v):
    N, c1, H, W = x.shape
    T, gc = guide.shape[1], guide.shape[2]
    HW = H * W
    c = cv1_w.shape[0] // 2
    c2 = cv2_w.shape[0]
    nh = attn_bias.shape[0]
    hc = c // nh
    bf16 = jnp.bfloat16

    conv_ws, conv_bs = [], []
    for w, g, b, m, v in (
            (m0_w1, m0_bn1_g, m0_bn1_b, m0_bn1_m, m0_bn1_v),
            (m0_w2, m0_bn2_g, m0_bn2_b, m0_bn2_m, m0_bn2_v),
            (m1_w1, m1_bn1_g, m1_bn1_b, m1_bn1_m, m1_bn1_v),
            (m1_w2, m1_bn2_g, m1_bn2_b, m1_bn2_m, m1_bn2_v),
            (m2_w1, m2_bn1_g, m2_bn1_b, m2_bn1_m, m2_bn1_v),
            (m2_w2, m2_bn2_g, m2_bn2_b, m2_bn2_m, m2_bn2_v),
            (proj_w, proj_bn_g, proj_bn_b, proj_bn_m, proj_bn_v)):
        s, bb = _fold_bn(g, b, m, v)
        conv_ws.append(w * s[:, None, None, None])
        conv_bs.append(bb)
    wall = jnp.transpose(jnp.stack(conv_ws), (0, 3, 1, 4, 2))
    wall = wall.reshape(7, 3, c, 3 * c).astype(bf16)
    ball = jnp.concatenate(
        [jnp.stack(conv_bs), jnp.zeros((1, c), jnp.float32)]
    ).reshape(8, c, 1)

    c1s, c1b = _fold_bn(cv1_bn_g, cv1_bn_b, cv1_bn_m, cv1_bn_v)
    c2s, c2b = _fold_bn(cv2_bn_g, cv2_bn_b, cv2_bn_m, cv2_bn_v)
    c1wf = (cv1_w.reshape(2 * c, c1) * c1s[:, None]).astype(bf16)
    c2wf = (cv2_w.reshape(c2, (3 + 3) * c) * c2s[:, None]).astype(bf16)
    ab_pad = jnp.zeros((8, 128), jnp.float32).at[:nh, :].set(
        attn_bias.reshape(nh, 1))
    masks = jnp.asarray(_edge_masks(H, W), dtype=bf16)

    operands = [x.reshape(N, c1, HW), guide, masks,
                c1wf, c1b.reshape(2 * c, 1), wall, ball,
                gl_w.astype(bf16), gl_b.reshape(1, -1), ab_pad,
                c2wf, c2b.reshape(c2, 1)]

    def body(x_ref, g_ref, m_ref, c1w, c1br, wall_ref, ball_ref,
             glw, glb, ab, c2w, c2br, o_ref):
        msk = m_ref[...]
        mxm, mxp, myt, myb = (msk[0:1], msk[1:2], msk[2:3], msk[3:4])

        def conv3(xb, i):
            x3 = jnp.concatenate(
                [_rotl(xb, -1, HW) * mxm, xb, _rotl(xb, 1, HW) * mxp], axis=0)
            acc = jnp.dot(wall_ref[i, 1], x3,
                          preferred_element_type=jnp.float32)
            acc = acc + jnp.dot(wall_ref[i, 0], _rotl(x3, -W, HW) * myt,
                                preferred_element_type=jnp.float32)
            acc = acc + jnp.dot(wall_ref[i, 2], _rotl(x3, W, HW) * myb,
                                preferred_element_type=jnp.float32)
            return acc + ball_ref[i]

        def silu(y):
            return y * jax.nn.sigmoid(y)

        xb = x_ref[0].astype(bf16)
        y = silu(jnp.dot(c1w[...], xb, preferred_element_type=jnp.float32)
                 + c1br[...])
        ys = [y[:c].astype(bf16)]
        cur = y[c:].astype(bf16)
        ys.append(cur)
        for i in range(3):
            t = silu(conv3(cur, 2 * i)).astype(bf16)
            cur = silu(conv3(t, 2 * i + 1)).astype(bf16)
            ys.append(cur)

        ge = lax.dot_general(g_ref[0].astype(bf16), glw[...],
                             (((1,), (1,)), ((), ())),
                             preferred_element_type=jnp.float32) + glb[...]
        geb = ge.astype(bf16)

        proj = conv3(cur, 6)
        rows = []
        for m in range(nh):
            s = jnp.dot(geb[:, m * hc:(m + 1) * hc],
                        cur[m * hc:(m + 1) * hc],
                        preferred_element_type=jnp.float32)
            rows.append(jnp.max(s, axis=0, keepdims=True))
        aw = jnp.concatenate(rows, axis=0) if nh > 1 else rows[0]
        gate = jax.nn.sigmoid(aw * (1.0 / float(hc) ** 0.5) + ab[0:nh, 0:1])
        att = [(proj[m * hc:(m + 1) * hc] * gate[m:m + 1]).astype(bf16)
               for m in range(nh)]
        ys.append(jnp.concatenate(att, axis=0) if nh > 1 else att[0])

        cat = jnp.concatenate(ys, axis=0)
        o_ref[0] = silu(
            jnp.dot(c2w[...], cat, preferred_element_type=jnp.float32)
            + c2br[...])

    fixed = lambda *shape: pl.BlockSpec(shape, lambda n: (0,) * len(shape))
    in_specs = [pl.BlockSpec((1, c1, HW), lambda n: (n, 0, 0)),
                pl.BlockSpec((1, T, gc), lambda n: (n, 0, 0)),
                fixed(8, HW),
                fixed(2 * c, c1), fixed(2 * c, 1),
                fixed(7, 3, c, 3 * c), fixed(8, c, 1),
                fixed(nh * hc, gc), fixed(1, nh * hc), fixed(8, 128),
                fixed(c2, (3 + 3) * c), fixed(c2, 1)]

    out = pl.pallas_call(
        body,
        out_shape=jax.ShapeDtypeStruct((N, c2, HW), jnp.float32),
        grid=(N,),
        in_specs=in_specs,
        out_specs=pl.BlockSpec((1, c2, HW), lambda n: (n, 0, 0)),
        compiler_params=pltpu.CompilerParams(
            dimension_semantics=("parallel",)),
    )(*operands)
    return out.reshape(N, c2, H, W)
